# Initial kernel scaffold; baseline (speedup 1.0000x reference)
import jax
import jax.numpy as jnp
from jax.experimental import pallas as pl


def kernel(x, edge_index, W1, W2):
    raise NotImplementedError("write your pallas kernel here")


# speedup vs baseline: 17.1425x; 17.1425x over previous
import functools

import jax
import jax.numpy as jnp
from jax import lax
from jax.experimental import pallas as pl
from jax.experimental.pallas import tpu as pltpu
from jax.experimental.pallas import tpu_sc as plsc

NC = 2
NS = 16
NW = NC * NS
L = 16
CH = 128


def _zero_rows(buf, nrows, ncols):
  z = jnp.zeros((L,), jnp.float32)

  def body(i, _):
    for jj in range(ncols // L):
      buf[i, pl.ds(jj * L, L)] = z
    return 0

  lax.fori_loop(0, nrows, body, 0)


def _fill_ones(buf, nrows, ncols):
  o = jnp.ones((L,), jnp.float32)

  def body(i, _):
    for jj in range(ncols // L):
      buf[i, pl.ds(jj * L, L)] = o
    return 0

  lax.fori_loop(0, nrows, body, 0)


def _make_deg_kernel(n, e):
  assert e % CH == 0 and n % (8 * NS) == 0
  nch = e // CH
  npt = n // NS
  zr = min(npt, 128)
  mesh = plsc.VectorSubcoreMesh(core_axis_name="c", subcore_axis_name="s")

  @functools.partial(
      pl.kernel,
      out_type=jax.ShapeDtypeStruct((NC, n, L), jnp.float32),
      mesh=mesh,
      compiler_params=pltpu.CompilerParams(use_tc_tiling_on_sc=False),
      scratch_types=[
          pltpu.VMEM_SHARED((n, L), jnp.float32),
          pltpu.VMEM((CH, L), jnp.float32),
          pltpu.VMEM((CH,), jnp.int32),
      ],
  )
  def deg_kernel(dst_hbm, degp_hbm, acc, buf, idx_v):
    c = lax.axis_index("c")
    s = lax.axis_index("s")
    w = c * NS + s
    row0 = s * npt

    _zero_rows(buf, zr, L)
    off = 0
    while off < npt:
      step = min(zr, npt - off)
      pltpu.sync_copy(buf.at[pl.ds(0, step)],
                      acc.at[pl.ds(row0 + off, step)])
      off += step
    plsc.subcore_barrier()

    _fill_ones(buf, CH, L)
    nj = (nch - w + NW - 1) // NW

    def body(j, _):
      base = (w + j * NW) * CH
      pltpu.sync_copy(dst_hbm.at[pl.ds(base, CH)], idx_v)
      pltpu.sync_copy(buf, acc.at[idx_v], add=True)
      return 0

    lax.fori_loop(0, nj, body, 0)
    plsc.subcore_barrier()
    pltpu.sync_copy(acc.at[pl.ds(row0, npt)],
                    degp_hbm.at[c, pl.ds(row0, npt)])

  return deg_kernel


def _make_agg_kernel(n, e, d):
  assert e % CH == 0 and n % (8 * NS) == 0 and d % L == 0
  nch = e // CH
  npt = n // NS
  zr = min(npt, 128)
  mesh = plsc.VectorSubcoreMesh(core_axis_name="c", subcore_axis_name="s")

  @functools.partial(
      pl.kernel,
      out_type=jax.ShapeDtypeStruct((NC, n, d), jnp.float32),
      mesh=mesh,
      compiler_params=pltpu.CompilerParams(use_tc_tiling_on_sc=False),
      scratch_types=[
          pltpu.VMEM_SHARED((n, d), jnp.float32),
          pltpu.VMEM((CH, d), jnp.float32),
          pltpu.VMEM((zr, d), jnp.float32),
          pltpu.VMEM((CH,), jnp.int32),
          pltpu.VMEM((CH,), jnp.int32),
          pltpu.SemaphoreType.DMA,
      ],
  )
  def agg_kernel(y_hbm, src_hbm, dst_hbm, aggp_hbm,
                 acc, rows_v, zbuf, sidx_v, didx_v, sem):
    c = lax.axis_index("c")
    s = lax.axis_index("s")
    w = c * NS + s
    row0 = s * npt

    _zero_rows(zbuf, zr, d)
    off = 0
    while off < npt:
      step = min(zr, npt - off)
      pltpu.sync_copy(zbuf.at[pl.ds(0, step)],
                      acc.at[pl.ds(row0 + off, step)])
      off += step
    plsc.subcore_barrier()

    nj = (nch - w + NW - 1) // NW

    def body(j, _):
      base = (w + j * NW) * CH
      pltpu.sync_copy(src_hbm.at[pl.ds(base, CH)], sidx_v)
      pltpu.sync_copy(dst_hbm.at[pl.ds(base, CH)], didx_v)
      pltpu.async_copy(y_hbm.at[sidx_v], rows_v, sem).wait()
      pltpu.sync_copy(rows_v, acc.at[didx_v], add=True)
      return 0

    lax.fori_loop(0, nj, body, 0)
    plsc.subcore_barrier()
    pltpu.sync_copy(acc.at[pl.ds(row0, npt)],
                    aggp_hbm.at[c, pl.ds(row0, npt)])

  return agg_kernel


def _tc_layer1(degp0_ref, degp1_ref, x_ref, w1_ref, dinv_ref, y1_ref):
  deg = degp0_ref[...] + degp1_ref[...] + 1.0
  dinv = lax.rsqrt(deg)
  dinv_ref[...] = dinv
  xw = jnp.dot(x_ref[...], w1_ref[...], preferred_element_type=jnp.float32)
  y1_ref[...] = xw * dinv[:, 0:1]


def _tc_layer2(aggp0_ref, aggp1_ref, y1_ref, dinv_ref, w2_ref, y2_ref):
  dv = dinv_ref[...][:, 0:1]
  h = jnp.tanh((aggp0_ref[...] + aggp1_ref[...] + y1_ref[...]) * dv)
  y2_ref[...] = jnp.dot(h, w2_ref[...],
                        preferred_element_type=jnp.float32) * dv


def _tc_final(aggp0_ref, aggp1_ref, y2_ref, dinv_ref, out_ref):
  dv = dinv_ref[...][:, 0:1]
  out_ref[...] = (aggp0_ref[...] + aggp1_ref[...] + y2_ref[...]) * dv


def kernel(x, edge_index, W1, W2):
  n, f_in = x.shape
  e = edge_index.shape[1]
  h = W1.shape[1]
  cdim = W2.shape[1]
  cpad = 128
  src = edge_index[0]
  dst = edge_index[1]
  W2p = jnp.zeros((h, cpad), jnp.float32).at[:, :cdim].set(W2)
  np_pad = -(-n // (8 * NS)) * (8 * NS)

  blk = 2000
  assert n % blk == 0
  grid = (n // blk,)
  row_spec = lambda width: pl.BlockSpec((blk, width), lambda i: (i, 0))
  full_spec = lambda r, ccol: pl.BlockSpec((r, ccol), lambda i: (0, 0))

  degp = _make_deg_kernel(np_pad, e)(dst)

  dinv, y1 = pl.pallas_call(
      _tc_layer1,
      grid=grid,
      in_specs=[row_spec(L), row_spec(L), row_spec(f_in), full_spec(f_in, h)],
      out_specs=[row_spec(L), row_spec(h)],
      out_shape=[
          jax.ShapeDtypeStruct((n, L), jnp.float32),
          jax.ShapeDtypeStruct((n, h), jnp.float32),
      ],
  )(degp[0], degp[1], x, W1)

  aggp1 = _make_agg_kernel(np_pad, e, h)(y1, src, dst)

  y2 = pl.pallas_call(
      _tc_layer2,
      grid=grid,
      in_specs=[row_spec(h), row_spec(h), row_spec(h), row_spec(L),
                full_spec(h, cpad)],
      out_specs=row_spec(cpad),
      out_shape=jax.ShapeDtypeStruct((n, cpad), jnp.float32),
  )(aggp1[0], aggp1[1], y1, dinv, W2p)

  aggp2 = _make_agg_kernel(np_pad, e, cpad)(y2, src, dst)

  out = pl.pallas_call(
      _tc_final,
      grid=grid,
      in_specs=[row_spec(cpad), row_spec(cpad), row_spec(cpad), row_spec(L)],
      out_specs=row_spec(cpad),
      out_shape=jax.ShapeDtypeStruct((n, cpad), jnp.float32),
  )(aggp2[0], aggp2[1], y2, dinv)

  return out[:, :cdim]


# speedup vs baseline: 34.2730x; 1.9993x over previous
import functools

import jax
import jax.numpy as jnp
from jax import lax
from jax.experimental import pallas as pl
from jax.experimental.pallas import tpu as pltpu
from jax.experimental.pallas import tpu_sc as plsc

NC = 2
NS = 16
NW = NC * NS
L = 16
CH = 80


def _zero_rows(buf, nrows, ncols):
  z = jnp.zeros((L,), jnp.float32)

  def body(i, _):
    for jj in range(ncols // L):
      buf[i, pl.ds(jj * L, L)] = z
    return 0

  lax.fori_loop(0, nrows, body, 0)


def _fill_ones(buf, nrows, ncols):
  o = jnp.ones((L,), jnp.float32)

  def body(i, _):
    for jj in range(ncols // L):
      buf[i, pl.ds(jj * L, L)] = o
    return 0

  lax.fori_loop(0, nrows, body, 0)


def _make_deg_kernel(n, e):
  assert e % (CH * NW) == 0 and n % (8 * NS) == 0
  nch = e // CH
  ncw = nch // NW
  npt = n // NS
  zr = min(npt, CH)
  mesh = plsc.VectorSubcoreMesh(core_axis_name="c", subcore_axis_name="s")

  @functools.partial(
      pl.kernel,
      out_type=jax.ShapeDtypeStruct((NC, n, L), jnp.float32),
      mesh=mesh,
      compiler_params=pltpu.CompilerParams(use_tc_tiling_on_sc=False),
      scratch_types=[
          pltpu.VMEM_SHARED((n, L), jnp.float32),
          pltpu.VMEM((CH, L), jnp.float32),
          pltpu.VMEM((ncw, CH), jnp.int32),
      ],
  )
  def deg_kernel(dst2_hbm, degp_hbm, acc, buf, didx_all):
    c = lax.axis_index("c")
    s = lax.axis_index("s")
    w = c * NS + s
    row0 = s * npt

    pltpu.sync_copy(dst2_hbm.at[pl.ds(w * ncw, ncw)], didx_all)

    _zero_rows(buf, zr, L)
    off = 0
    while off < npt:
      step = min(zr, npt - off)
      pltpu.sync_copy(buf.at[pl.ds(0, step)],
                      acc.at[pl.ds(row0 + off, step)])
      off += step
    plsc.subcore_barrier()

    _fill_ones(buf, CH, L)

    def body(j, _):
      pltpu.sync_copy(buf, acc.at[didx_all.at[j]], add=True)
      return 0

    lax.fori_loop(0, ncw, body, 0)
    plsc.subcore_barrier()
    pltpu.sync_copy(acc.at[pl.ds(row0, npt)],
                    degp_hbm.at[c, pl.ds(row0, npt)])

  return deg_kernel


def _make_agg_kernel(n, e, d):
  assert e % (CH * NW) == 0 and n % (8 * NS) == 0 and d % L == 0
  nch = e // CH
  ncw = nch // NW
  npt = n // NS
  zr = min(npt, CH)
  ring = 4 if d <= 64 else 2
  assert ncw >= ring
  mesh = plsc.VectorSubcoreMesh(core_axis_name="c", subcore_axis_name="s")

  @functools.partial(
      pl.kernel,
      out_type=jax.ShapeDtypeStruct((NC, n, d), jnp.float32),
      mesh=mesh,
      compiler_params=pltpu.CompilerParams(use_tc_tiling_on_sc=False),
      scratch_types=[
          pltpu.VMEM_SHARED((n, d), jnp.float32),
          pltpu.VMEM((ring, CH, d), jnp.float32),
          pltpu.VMEM((ncw, CH), jnp.int32),
          pltpu.VMEM((ncw, CH), jnp.int32),
          pltpu.SemaphoreType.DMA((ring,)),
      ],
  )
  def agg_kernel(y_hbm, src2_hbm, dst2_hbm, aggp_hbm,
                 acc, rows_v, sidx_all, didx_all, gsem):
    c = lax.axis_index("c")
    s = lax.axis_index("s")
    w = c * NS + s
    row0 = s * npt

    pltpu.sync_copy(src2_hbm.at[pl.ds(w * ncw, ncw)], sidx_all)
    pltpu.sync_copy(dst2_hbm.at[pl.ds(w * ncw, ncw)], didx_all)

    zslot = rows_v.at[0]
    _zero_rows(zslot, zr, d)
    off = 0
    while off < npt:
      step = min(zr, npt - off)
      pltpu.sync_copy(zslot.at[pl.ds(0, step)],
                      acc.at[pl.ds(row0 + off, step)])
      off += step
    plsc.subcore_barrier()

    for jj in range(ring):
      pltpu.async_copy(y_hbm.at[sidx_all.at[jj]], rows_v.at[jj],
                       gsem.at[jj])

    def body(j, _):
      rb = j % ring
      pltpu.make_async_copy(y_hbm.at[sidx_all.at[j]], rows_v.at[rb],
                            gsem.at[rb]).wait()
      pltpu.sync_copy(rows_v.at[rb], acc.at[didx_all.at[j]], add=True)
      pltpu.async_copy(y_hbm.at[sidx_all.at[j + ring]], rows_v.at[rb],
                       gsem.at[rb])
      return 0

    lax.fori_loop(0, ncw - ring, body, 0)

    def tail(j, _):
      rb = j % ring
      pltpu.make_async_copy(y_hbm.at[sidx_all.at[j]], rows_v.at[rb],
                            gsem.at[rb]).wait()
      pltpu.sync_copy(rows_v.at[rb], acc.at[didx_all.at[j]], add=True)
      return 0

    lax.fori_loop(ncw - ring, ncw, tail, 0)
    plsc.subcore_barrier()
    pltpu.sync_copy(acc.at[pl.ds(row0, npt)],
                    aggp_hbm.at[c, pl.ds(row0, npt)])

  return agg_kernel


def _tc_layer1(degp0_ref, degp1_ref, x_ref, w1_ref, dinv_ref, y1_ref):
  deg = degp0_ref[...] + degp1_ref[...] + 1.0
  dinv = lax.rsqrt(deg)
  dinv_ref[...] = dinv
  xw = jnp.dot(x_ref[...], w1_ref[...], preferred_element_type=jnp.float32)
  y1_ref[...] = xw * dinv[:, 0:1]


def _tc_layer2(aggp0_ref, aggp1_ref, y1_ref, dinv_ref, w2_ref, y2_ref):
  dv = dinv_ref[...][:, 0:1]
  h = jnp.tanh((aggp0_ref[...] + aggp1_ref[...] + y1_ref[...]) * dv)
  y2_ref[...] = jnp.dot(h, w2_ref[...],
                        preferred_element_type=jnp.float32) * dv


def _tc_final(aggp0_ref, aggp1_ref, y2_ref, dinv_ref, out_ref):
  dv = dinv_ref[...][:, 0:1]
  out_ref[...] = (aggp0_ref[...] + aggp1_ref[...] + y2_ref[...]) * dv


def kernel(x, edge_index, W1, W2):
  n, f_in = x.shape
  e = edge_index.shape[1]
  h = W1.shape[1]
  cdim = W2.shape[1]
  cpad = 128
  assert e % CH == 0
  src2 = edge_index[0].reshape(e // CH, CH)
  dst2 = edge_index[1].reshape(e // CH, CH)
  W2p = jnp.zeros((h, cpad), jnp.float32).at[:, :cdim].set(W2)
  np_pad = -(-n // (8 * NS)) * (8 * NS)

  blk = 2000
  assert n % blk == 0
  grid = (n // blk,)
  row_spec = lambda width: pl.BlockSpec((blk, width), lambda i: (i, 0))
  full_spec = lambda r, ccol: pl.BlockSpec((r, ccol), lambda i: (0, 0))

  degp = _make_deg_kernel(np_pad, e)(dst2)

  dinv, y1 = pl.pallas_call(
      _tc_layer1,
      grid=grid,
      in_specs=[row_spec(L), row_spec(L), row_spec(f_in), full_spec(f_in, h)],
      out_specs=[row_spec(L), row_spec(h)],
      out_shape=[
          jax.ShapeDtypeStruct((n, L), jnp.float32),
          jax.ShapeDtypeStruct((n, h), jnp.float32),
      ],
  )(degp[0], degp[1], x, W1)

  aggp1 = _make_agg_kernel(np_pad, e, h)(y1, src2, dst2)

  y2 = pl.pallas_call(
      _tc_layer2,
      grid=grid,
      in_specs=[row_spec(h), row_spec(h), row_spec(h), row_spec(L),
                full_spec(h, cpad)],
      out_specs=row_spec(cpad),
      out_shape=jax.ShapeDtypeStruct((n, cpad), jnp.float32),
  )(aggp1[0], aggp1[1], y1, dinv, W2p)

  aggp2 = _make_agg_kernel(np_pad, e, cpad)(y2, src2, dst2)

  out = pl.pallas_call(
      _tc_final,
      grid=grid,
      in_specs=[row_spec(cpad), row_spec(cpad), row_spec(cpad), row_spec(L)],
      out_specs=row_spec(cpad),
      out_shape=jax.ShapeDtypeStruct((n, cpad), jnp.float32),
  )(aggp2[0], aggp2[1], y2, dinv)

  return out[:, :cdim]
